# Initial kernel scaffold; baseline (speedup 1.0000x reference)
import functools

import jax
import jax.numpy as jnp
from jax import lax
from jax.experimental import pallas as pl
from jax.experimental.pallas import tpu as pltpu
from jax.experimental.pallas import tpu_sc as plsc

L = 16


def _round_up(v, m):
    return ((v + m - 1) // m) * m


def _make_k1(NPAD, KCH):
    KH = KCH // 2
    W = NPAD // 16
    mesh = plsc.VectorSubcoreMesh(core_axis_name="c", subcore_axis_name="s")

    @functools.partial(
        pl.kernel,
        out_type=jax.ShapeDtypeStruct((2, 2, NPAD), jnp.float32),
        mesh=mesh,
        scratch_types=[
            pltpu.VMEM((KH, 128), jnp.int32),
            pltpu.VMEM((KH, 128), jnp.int32),
            pltpu.VMEM((NPAD,), jnp.float32),
            pltpu.VMEM((NPAD,), jnp.float32),
            pltpu.VMEM((16, W), jnp.float32),
            pltpu.VMEM((W,), jnp.float32),
            pltpu.VMEM_SHARED((16, 2, NPAD), jnp.float32),
        ],
    )
    def k1(src_hbm, dst_hbm, out_hbm, srcb, dstb, cntf, cntu, red, acc, shared):
        c = lax.axis_index("c")
        s = lax.axis_index("s")
        pltpu.sync_copy(src_hbm.at[s, pl.ds(c * KH, KH)], srcb)
        pltpu.sync_copy(dst_hbm.at[s, pl.ds(c * KH, KH)], dstb)

        zeros = jnp.zeros((L,), jnp.float32)

        def zero_body(i, _):
            cntf[pl.ds(i * L, L)] = zeros
            cntu[pl.ds(i * L, L)] = zeros
            return _

        lax.fori_loop(0, NPAD // L, zero_body, None)

        ones = jnp.ones((L,), jnp.float32)

        def cnt_body(k, _):
            for j in range(128 // L):
                sv = srcb[k, pl.ds(j * L, L)]
                dv = dstb[k, pl.ds(j * L, L)]
                plsc.addupdate_scatter(cntf, [dv], ones)
                plsc.addupdate_scatter(cntu, [sv], ones)
            return _

        lax.fori_loop(0, KH, cnt_body, None)

        pltpu.sync_copy(cntf, shared.at[s, 0])
        pltpu.sync_copy(cntu, shared.at[s, 1])
        plsc.subcore_barrier()

        off = s * W
        for which in range(2):
            for t in range(16):
                pltpu.sync_copy(shared.at[t, which, pl.ds(off, W)], red.at[t])

            def red_body(j, _):
                v = red[0, pl.ds(j * L, L)]
                for t in range(1, 16):
                    v = v + red[t, pl.ds(j * L, L)]
                acc[pl.ds(j * L, L)] = v
                return _

            lax.fori_loop(0, W // L, red_body, None)
            pltpu.sync_copy(acc, out_hbm.at[c, which, pl.ds(off, W)])

    return k1


def _make_k3(NPAD, KCH):
    ZR = NPAD // 16
    mesh = plsc.VectorSubcoreMesh(core_axis_name="c", subcore_axis_name="s")

    @functools.partial(
        pl.kernel,
        out_type=jax.ShapeDtypeStruct((2, NPAD, 128), jnp.float32),
        mesh=mesh,
        scratch_types=[
            pltpu.VMEM((KCH, 128), jnp.int32),
            pltpu.VMEM((KCH, 128), jnp.int32),
            pltpu.VMEM((128, 128), jnp.float32),
            pltpu.VMEM_SHARED((NPAD, 128), jnp.float32),
            pltpu.SemaphoreType.DMA,
        ],
    )
    def k3(table_hbm, gidx_hbm, sidx_hbm, out_hbm, gbuf, sbuf, rows, acc, sem):
        c = lax.axis_index("c")
        s = lax.axis_index("s")
        pltpu.sync_copy(gidx_hbm.at[c, s], gbuf)
        pltpu.sync_copy(sidx_hbm.at[s], sbuf)

        zeros = jnp.zeros((L,), jnp.float32)

        def zrow_body(i, _):
            for j in range(128 // L):
                rows[i, pl.ds(j * L, L)] = zeros
            return _

        lax.fori_loop(0, 128, zrow_body, None)
        for r in range(ZR // 128):
            pltpu.sync_copy(rows, acc.at[pl.ds(s * ZR + r * 128, 128)])
        plsc.subcore_barrier()

        def main_body(k, _):
            pltpu.async_copy(table_hbm.at[gbuf.at[k]], rows, sem).wait()
            pltpu.sync_copy(rows, acc.at[sbuf.at[k]], add=True)
            return _

        lax.fori_loop(0, KCH, main_body, None)
        plsc.subcore_barrier()
        pltpu.sync_copy(acc.at[pl.ds(s * ZR, ZR)], out_hbm.at[c, pl.ds(s * ZR, ZR)])

    return k3


def _make_k5(NPAD, KCH, N):
    W = NPAD // 32
    mesh = plsc.VectorSubcoreMesh(core_axis_name="c", subcore_axis_name="s")

    @functools.partial(
        pl.kernel,
        out_type=jax.ShapeDtypeStruct((NPAD,), jnp.float32),
        mesh=mesh,
        scratch_types=[
            pltpu.VMEM((KCH, 128), jnp.int32),
            pltpu.VMEM((KCH, 128), jnp.int32),
            pltpu.VMEM((NPAD,), jnp.float32),
            pltpu.VMEM((NPAD,), jnp.float32),
            pltpu.VMEM((16, W), jnp.float32),
            pltpu.VMEM((W,), jnp.float32),
            pltpu.VMEM((W,), jnp.float32),
            pltpu.VMEM((L,), jnp.float32),
            pltpu.VMEM_SHARED((16, NPAD), jnp.float32),
        ],
    )
    def k5(sp_hbm, dinv_hbm, src_hbm, dst_hbm, bo_hbm, out_hbm,
           srcb, dstb, spbuf, tbuf, red, dvb, obuf, bob, shared):
        c = lax.axis_index("c")
        s = lax.axis_index("s")
        pltpu.sync_copy(sp_hbm, spbuf)
        pltpu.sync_copy(src_hbm.at[s], srcb)
        pltpu.sync_copy(dst_hbm.at[s], dstb)
        pltpu.sync_copy(bo_hbm, bob)

        zeros = jnp.zeros((L,), jnp.float32)

        def zero_body(i, _):
            tbuf[pl.ds(i * L, L)] = zeros
            return _

        lax.fori_loop(0, NPAD // L, zero_body, None)

        def scat_body(k, _):
            for j in range(128 // L):
                sv = srcb[k, pl.ds(j * L, L)]
                dv = dstb[k, pl.ds(j * L, L)]
                g = plsc.load_gather(spbuf, [sv])
                plsc.addupdate_scatter(tbuf, [dv], g)
            return _

        lax.fori_loop(0, KCH, scat_body, None)

        pltpu.sync_copy(tbuf, shared.at[s])
        plsc.subcore_barrier()

        off = (c * 16 + s) * W
        for t in range(16):
            pltpu.sync_copy(shared.at[t, pl.ds(off, W)], red.at[t])
        pltpu.sync_copy(dinv_hbm.at[pl.ds(off, W)], dvb)

        bo = bob[pl.ds(0, L)]
        one = jnp.ones((L,), jnp.float32)

        def fin_body(j, _):
            v = red[0, pl.ds(j * L, L)]
            for t in range(1, 16):
                v = v + red[t, pl.ds(j * L, L)]
            z = dvb[pl.ds(j * L, L)] * (v + spbuf[pl.ds(off + j * L, L)]) + bo
            obuf[pl.ds(j * L, L)] = one / (one + jnp.exp(-z))
            return _

        lax.fori_loop(0, W // L, fin_body, None)
        pltpu.sync_copy(obuf, out_hbm.at[pl.ds(off, W)])

    return k5


def _k2_body(HID, x_ref, wt_ref, cf0_ref, cf1_ref, cu0_ref, cu1_ref,
             yf_ref, yu_ref, dvf_ref, dvu_ref):
    HC = HID // 2
    y = jnp.dot(x_ref[...], wt_ref[...], preferred_element_type=jnp.float32)
    dinvf = lax.rsqrt(cf0_ref[...] + cf1_ref[...] + 1.0)
    dinvu = lax.rsqrt(cu0_ref[...] + cu1_ref[...] + 1.0)
    hf = y[:, :HID] * dinvf
    hu = y[:, HID:] * dinvu
    yf_ref[0] = hf[:, :HC]
    yf_ref[1] = hf[:, HC:]
    yu_ref[0] = hu[:, :HC]
    yu_ref[1] = hu[:, HC:]
    dvf_ref[...] = dinvf
    dvu_ref[...] = dinvu


def _k4_body(N, B, segf_ref, segu_ref, yf_ref, yu_ref, dvf_ref, dvu_ref,
             bf_ref, bu_ref, wfc_ref, bfc_ref, wo_ref, sp_ref):
    dvf = dvf_ref[...]
    hf = jnp.concatenate([segf_ref[0] + yf_ref[0], segf_ref[1] + yf_ref[1]], axis=1)
    hf = jnp.maximum(dvf * hf + bf_ref[...], 0.0)
    hu = jnp.concatenate([segu_ref[0] + yu_ref[0], segu_ref[1] + yu_ref[1]], axis=1)
    hu = jnp.maximum(dvu_ref[...] * hu + bu_ref[...], 0.0)
    hcat = jnp.concatenate([hf, hu], axis=1)
    h2 = jnp.dot(hcat, wfc_ref[...], preferred_element_type=jnp.float32)
    h2 = jnp.maximum(h2 + bfc_ref[...], 0.0)
    sv = jnp.dot(h2, wo_ref[...], preferred_element_type=jnp.float32)
    row = pl.program_id(0) * B + lax.broadcasted_iota(jnp.int32, (B, 1), 0)
    sp_ref[...] = jnp.where(row < N, dvf * sv, 0.0)


def kernel(x, edge_index, Wf, bf, Wu, bu, Wfc, bfc, Wo, bo):
    N, IN = x.shape
    HID = Wf.shape[0]
    HC = HID // 2
    E = edge_index.shape[1]
    NPAD = _round_up(N + 8, 2048)
    EPAD = _round_up(E, 32 * 128)
    EPT = EPAD // 16
    KCH = EPT // 128
    B = 2048
    GRID = NPAD // B

    src = edge_index[0].astype(jnp.int32)
    dst = edge_index[1].astype(jnp.int32)
    pad = jnp.full((EPAD - E,), N, jnp.int32)
    srcp = jnp.concatenate([src, pad])
    dstp = jnp.concatenate([dst, pad])
    sr3 = srcp.reshape(16, KCH, 128)
    dr3 = dstp.reshape(16, KCH, 128)
    g_f = jnp.stack([srcp, srcp + NPAD]).reshape(2, 16, KCH, 128)
    g_u = jnp.stack([dstp, dstp + NPAD]).reshape(2, 16, KCH, 128)

    xp = jnp.pad(x, ((0, NPAD - N), (0, 0)))
    WcatT = jnp.concatenate([Wf, Wu], axis=0).T
    WfcT = Wfc.T
    WoT = Wo.T

    counts = _make_k1(NPAD, KCH)(sr3, dr3)
    cf0 = counts[0, 0].reshape(NPAD, 1)
    cf1 = counts[1, 0].reshape(NPAD, 1)
    cu0 = counts[0, 1].reshape(NPAD, 1)
    cu1 = counts[1, 1].reshape(NPAD, 1)

    col = pl.BlockSpec((B, 1), lambda i: (i, 0))
    tab = pl.BlockSpec((2, B, HC), lambda i: (0, i, 0))
    yf, yu, dvf, dvu = pl.pallas_call(
        functools.partial(_k2_body, HID),
        grid=(GRID,),
        in_specs=[
            pl.BlockSpec((B, IN), lambda i: (i, 0)),
            pl.BlockSpec((IN, 2 * HID), lambda i: (0, 0)),
            col, col, col, col,
        ],
        out_specs=[tab, tab, col, col],
        out_shape=[
            jax.ShapeDtypeStruct((2, NPAD, HC), jnp.float32),
            jax.ShapeDtypeStruct((2, NPAD, HC), jnp.float32),
            jax.ShapeDtypeStruct((NPAD, 1), jnp.float32),
            jax.ShapeDtypeStruct((NPAD, 1), jnp.float32),
        ],
    )(xp, WcatT, cf0, cf1, cu0, cu1)

    k3 = _make_k3(NPAD, KCH)
    segf = k3(yf.reshape(2 * NPAD, HC), g_f, dr3)
    segu = k3(yu.reshape(2 * NPAD, HC), g_u, sr3)

    sp = pl.pallas_call(
        functools.partial(_k4_body, N, B),
        grid=(GRID,),
        in_specs=[
            tab, tab, tab, tab, col, col,
            pl.BlockSpec((1, HID), lambda i: (0, 0)),
            pl.BlockSpec((1, HID), lambda i: (0, 0)),
            pl.BlockSpec((2 * HID, HID), lambda i: (0, 0)),
            pl.BlockSpec((1, HID), lambda i: (0, 0)),
            pl.BlockSpec((HID, 1), lambda i: (0, 0)),
        ],
        out_specs=col,
        out_shape=jax.ShapeDtypeStruct((NPAD, 1), jnp.float32),
    )(segf, segu, yf, yu, dvf, dvu,
      bf.reshape(1, HID), bu.reshape(1, HID), WfcT, bfc.reshape(1, HID), WoT)

    bo16 = jnp.broadcast_to(bo, (L,)).astype(jnp.float32)
    out = _make_k5(NPAD, KCH, N)(
        sp.reshape(NPAD), dvf.reshape(NPAD), sr3, dr3, bo16)
    return out[:N].reshape(N, 1)


# speedup vs baseline: 9.8443x; 9.8443x over previous
import functools

import jax
import jax.numpy as jnp
from jax import lax
from jax.experimental import pallas as pl
from jax.experimental.pallas import tpu as pltpu
from jax.experimental.pallas import tpu_sc as plsc

L = 16


def _round_up(v, m):
    return ((v + m - 1) // m) * m


def _make_k1(NPAD, KCH):
    KH = KCH // 2
    W = NPAD // 16
    mesh = plsc.VectorSubcoreMesh(core_axis_name="c", subcore_axis_name="s")

    @functools.partial(
        pl.kernel,
        out_type=jax.ShapeDtypeStruct((2, 2, NPAD), jnp.float32),
        mesh=mesh,
        compiler_params=pltpu.CompilerParams(needs_layout_passes=False),
        scratch_types=[
            pltpu.VMEM((KH, 128), jnp.int32),
            pltpu.VMEM((KH, 128), jnp.int32),
            pltpu.VMEM((NPAD,), jnp.float32),
            pltpu.VMEM((NPAD,), jnp.float32),
            pltpu.VMEM((16 * W,), jnp.float32),
            pltpu.VMEM((W,), jnp.float32),
            pltpu.VMEM_SHARED((2 * 16 * NPAD,), jnp.float32),
        ],
    )
    def k1(src_hbm, dst_hbm, out_hbm, srcb, dstb, cntf, cntu, red, acc, shared):
        c = lax.axis_index("c")
        s = lax.axis_index("s")
        pltpu.sync_copy(src_hbm.at[s, pl.ds(c * KH, KH)], srcb)
        pltpu.sync_copy(dst_hbm.at[s, pl.ds(c * KH, KH)], dstb)

        zeros = jnp.zeros((L,), jnp.float32)

        def zero_body(i, _):
            cntf[pl.ds(i * L, L)] = zeros
            cntu[pl.ds(i * L, L)] = zeros
            return _

        lax.fori_loop(0, NPAD // L, zero_body, None)

        ones = jnp.ones((L,), jnp.float32)

        def cnt_body(k, _):
            for j in range(128 // L):
                sv = srcb[k, pl.ds(j * L, L)]
                dv = dstb[k, pl.ds(j * L, L)]
                plsc.addupdate_scatter(cntf, [dv], ones)
                plsc.addupdate_scatter(cntu, [sv], ones)
            return _

        lax.fori_loop(0, KH, cnt_body, None)

        pltpu.sync_copy(cntf, shared.at[pl.ds((2 * s) * NPAD, NPAD)])
        pltpu.sync_copy(cntu, shared.at[pl.ds((2 * s + 1) * NPAD, NPAD)])
        plsc.subcore_barrier()

        off = s * W
        for which in range(2):
            for t in range(16):
                pltpu.sync_copy(
                    shared.at[pl.ds((2 * t + which) * NPAD + off, W)],
                    red.at[pl.ds(t * W, W)])

            def red_body(j, _):
                v = red[pl.ds(j * L, L)]
                for t in range(1, 16):
                    v = v + red[pl.ds(t * W + j * L, L)]
                acc[pl.ds(j * L, L)] = v
                return _

            lax.fori_loop(0, W // L, red_body, None)
            pltpu.sync_copy(acc, out_hbm.at[c, which, pl.ds(off, W)])

    return k1


def _make_k3(NPAD, KCH):
    ZR = NPAD // 16
    mesh = plsc.VectorSubcoreMesh(core_axis_name="c", subcore_axis_name="s")

    @functools.partial(
        pl.kernel,
        out_type=jax.ShapeDtypeStruct((2, NPAD, 128), jnp.float32),
        mesh=mesh,
        compiler_params=pltpu.CompilerParams(needs_layout_passes=False),
        scratch_types=[
            pltpu.VMEM((KCH, 128), jnp.int32),
            pltpu.VMEM((KCH, 128), jnp.int32),
            pltpu.VMEM((128, 128), jnp.float32),
            pltpu.VMEM_SHARED((NPAD, 128), jnp.float32),
            pltpu.SemaphoreType.DMA,
        ],
    )
    def k3(table_hbm, gidx_hbm, sidx_hbm, out_hbm, gbuf, sbuf, rows, acc, sem):
        c = lax.axis_index("c")
        s = lax.axis_index("s")
        pltpu.sync_copy(gidx_hbm.at[c, s], gbuf)
        pltpu.sync_copy(sidx_hbm.at[s], sbuf)

        zeros = jnp.zeros((L,), jnp.float32)

        def zrow_body(i, _):
            for j in range(128 // L):
                rows[i, pl.ds(j * L, L)] = zeros
            return _

        lax.fori_loop(0, 128, zrow_body, None)
        for r in range(ZR // 128):
            pltpu.sync_copy(rows, acc.at[pl.ds(s * ZR + r * 128, 128)])
        plsc.subcore_barrier()

        def main_body(k, _):
            pltpu.async_copy(table_hbm.at[gbuf.at[k]], rows, sem).wait()
            pltpu.sync_copy(rows, acc.at[sbuf.at[k]], add=True)
            return _

        lax.fori_loop(0, KCH, main_body, None)
        plsc.subcore_barrier()
        pltpu.sync_copy(acc.at[pl.ds(s * ZR, ZR)], out_hbm.at[c, pl.ds(s * ZR, ZR)])

    return k3


def _make_k5(NPAD, KCH, N):
    W = NPAD // 16
    mesh = plsc.VectorSubcoreMesh(core_axis_name="c", subcore_axis_name="s")

    @functools.partial(
        pl.kernel,
        out_type=jax.ShapeDtypeStruct((NPAD,), jnp.float32),
        mesh=mesh,
        compiler_params=pltpu.CompilerParams(needs_layout_passes=False),
        scratch_types=[
            pltpu.VMEM((KCH, 128), jnp.int32),
            pltpu.VMEM((KCH, 128), jnp.int32),
            pltpu.VMEM((NPAD,), jnp.float32),
            pltpu.VMEM((NPAD,), jnp.float32),
            pltpu.VMEM((16 * W,), jnp.float32),
            pltpu.VMEM((W,), jnp.float32),
            pltpu.VMEM((W,), jnp.float32),
            pltpu.VMEM((L,), jnp.float32),
            pltpu.VMEM_SHARED((16 * NPAD,), jnp.float32),
        ],
    )
    def k5(sp_hbm, dinv_hbm, src_hbm, dst_hbm, bo_hbm, out_hbm,
           srcb, dstb, spbuf, tbuf, red, dvb, obuf, bob, shared):
        c = lax.axis_index("c")
        s = lax.axis_index("s")

        @pl.when(c == 0)
        def _sc0():
            pltpu.sync_copy(sp_hbm, spbuf)
            pltpu.sync_copy(src_hbm.at[s], srcb)
            pltpu.sync_copy(dst_hbm.at[s], dstb)
            pltpu.sync_copy(bo_hbm, bob)

            zeros = jnp.zeros((L,), jnp.float32)

            def zero_body(i, _):
                tbuf[pl.ds(i * L, L)] = zeros
                return _

            lax.fori_loop(0, NPAD // L, zero_body, None)

            def scat_body(k, _):
                for j in range(128 // L):
                    sv = srcb[k, pl.ds(j * L, L)]
                    dv = dstb[k, pl.ds(j * L, L)]
                    g = plsc.load_gather(spbuf, [sv])
                    plsc.addupdate_scatter(tbuf, [dv], g)
                return _

            lax.fori_loop(0, KCH, scat_body, None)

            pltpu.sync_copy(tbuf, shared.at[pl.ds(s * NPAD, NPAD)])
            plsc.subcore_barrier()

            off = s * W
            for t in range(16):
                pltpu.sync_copy(shared.at[pl.ds(t * NPAD + off, W)],
                                red.at[pl.ds(t * W, W)])
            pltpu.sync_copy(dinv_hbm.at[pl.ds(off, W)], dvb)

            bo = bob[pl.ds(0, L)]
            one = jnp.ones((L,), jnp.float32)

            def fin_body(j, _):
                v = red[pl.ds(j * L, L)]
                for t in range(1, 16):
                    v = v + red[pl.ds(t * W + j * L, L)]
                z = dvb[pl.ds(j * L, L)] * (v + spbuf[pl.ds(off + j * L, L)]) + bo
                obuf[pl.ds(j * L, L)] = one / (one + jnp.exp(-z))
                return _

            lax.fori_loop(0, W // L, fin_body, None)
            pltpu.sync_copy(obuf, out_hbm.at[pl.ds(off, W)])

    return k5


def _k2_body(HID, x_ref, wt_ref, cf0_ref, cf1_ref, cu0_ref, cu1_ref,
             yf_ref, yu_ref, dvf_ref, dvu_ref):
    HC = HID // 2
    y = jnp.dot(x_ref[...], wt_ref[...], preferred_element_type=jnp.float32)
    dinvf = lax.rsqrt(cf0_ref[...] + cf1_ref[...] + 1.0)
    dinvu = lax.rsqrt(cu0_ref[...] + cu1_ref[...] + 1.0)
    hf = y[:, :HID] * dinvf
    hu = y[:, HID:] * dinvu
    yf_ref[0] = hf[:, :HC]
    yf_ref[1] = hf[:, HC:]
    yu_ref[0] = hu[:, :HC]
    yu_ref[1] = hu[:, HC:]
    dvf_ref[...] = dinvf
    dvu_ref[...] = dinvu


def _k4_body(N, B, segf_ref, segu_ref, yf_ref, yu_ref, dvf_ref, dvu_ref,
             bf_ref, bu_ref, wfc_ref, bfc_ref, wo_ref, sp_ref):
    dvf = dvf_ref[...]
    hf = jnp.concatenate([segf_ref[0] + yf_ref[0], segf_ref[1] + yf_ref[1]], axis=1)
    hf = jnp.maximum(dvf * hf + bf_ref[...], 0.0)
    hu = jnp.concatenate([segu_ref[0] + yu_ref[0], segu_ref[1] + yu_ref[1]], axis=1)
    hu = jnp.maximum(dvu_ref[...] * hu + bu_ref[...], 0.0)
    hcat = jnp.concatenate([hf, hu], axis=1)
    h2 = jnp.dot(hcat, wfc_ref[...], preferred_element_type=jnp.float32)
    h2 = jnp.maximum(h2 + bfc_ref[...], 0.0)
    sv = jnp.dot(h2, wo_ref[...], preferred_element_type=jnp.float32)
    row = pl.program_id(0) * B + lax.broadcasted_iota(jnp.int32, (B, 1), 0)
    sp_ref[...] = jnp.where(row < N, dvf * sv, 0.0)


def kernel(x, edge_index, Wf, bf, Wu, bu, Wfc, bfc, Wo, bo):
    N, IN = x.shape
    HID = Wf.shape[0]
    HC = HID // 2
    E = edge_index.shape[1]
    NPAD = _round_up(N + 8, 2048)
    EPAD = _round_up(E, 32 * 128)
    EPT = EPAD // 16
    KCH = EPT // 128
    B = 2048
    GRID = NPAD // B

    src = edge_index[0].astype(jnp.int32)
    dst = edge_index[1].astype(jnp.int32)
    pad = jnp.full((EPAD - E,), N, jnp.int32)
    srcp = jnp.concatenate([src, pad])
    dstp = jnp.concatenate([dst, pad])
    sr3 = srcp.reshape(16, KCH, 128)
    dr3 = dstp.reshape(16, KCH, 128)
    g_f = jnp.stack([srcp, srcp + NPAD]).reshape(2, 16, KCH, 128)
    g_u = jnp.stack([dstp, dstp + NPAD]).reshape(2, 16, KCH, 128)

    xp = jnp.pad(x, ((0, NPAD - N), (0, 0)))
    WcatT = jnp.concatenate([Wf, Wu], axis=0).T
    WfcT = Wfc.T
    WoT = Wo.T

    counts = _make_k1(NPAD, KCH)(sr3, dr3)
    cf0 = counts[0, 0].reshape(NPAD, 1)
    cf1 = counts[1, 0].reshape(NPAD, 1)
    cu0 = counts[0, 1].reshape(NPAD, 1)
    cu1 = counts[1, 1].reshape(NPAD, 1)

    col = pl.BlockSpec((B, 1), lambda i: (i, 0))
    tab = pl.BlockSpec((2, B, HC), lambda i: (0, i, 0))
    yf, yu, dvf, dvu = pl.pallas_call(
        functools.partial(_k2_body, HID),
        grid=(GRID,),
        in_specs=[
            pl.BlockSpec((B, IN), lambda i: (i, 0)),
            pl.BlockSpec((IN, 2 * HID), lambda i: (0, 0)),
            col, col, col, col,
        ],
        out_specs=[tab, tab, col, col],
        out_shape=[
            jax.ShapeDtypeStruct((2, NPAD, HC), jnp.float32),
            jax.ShapeDtypeStruct((2, NPAD, HC), jnp.float32),
            jax.ShapeDtypeStruct((NPAD, 1), jnp.float32),
            jax.ShapeDtypeStruct((NPAD, 1), jnp.float32),
        ],
    )(xp, WcatT, cf0, cf1, cu0, cu1)

    k3 = _make_k3(NPAD, KCH)
    segf = k3(yf.reshape(2 * NPAD, HC), g_f, dr3)
    segu = k3(yu.reshape(2 * NPAD, HC), g_u, sr3)

    sp = pl.pallas_call(
        functools.partial(_k4_body, N, B),
        grid=(GRID,),
        in_specs=[
            tab, tab, tab, tab, col, col,
            pl.BlockSpec((1, HID), lambda i: (0, 0)),
            pl.BlockSpec((1, HID), lambda i: (0, 0)),
            pl.BlockSpec((2 * HID, HID), lambda i: (0, 0)),
            pl.BlockSpec((1, HID), lambda i: (0, 0)),
            pl.BlockSpec((HID, 1), lambda i: (0, 0)),
        ],
        out_specs=col,
        out_shape=jax.ShapeDtypeStruct((NPAD, 1), jnp.float32),
    )(segf, segu, yf, yu, dvf, dvu,
      bf.reshape(1, HID), bu.reshape(1, HID), WfcT, bfc.reshape(1, HID), WoT)

    bo16 = jnp.broadcast_to(bo, (L,)).astype(jnp.float32)
    out = _make_k5(NPAD, KCH, N)(
        sp.reshape(NPAD), dvf.reshape(NPAD), sr3, dr3, bo16)
    return out[:N].reshape(N, 1)


# speedup vs baseline: 11.2357x; 1.1413x over previous
import functools

import jax
import jax.numpy as jnp
from jax import lax
from jax.experimental import pallas as pl
from jax.experimental.pallas import tpu as pltpu
from jax.experimental.pallas import tpu_sc as plsc

L = 16


def _round_up(v, m):
    return ((v + m - 1) // m) * m


def _make_k1(NPAD, KCH):
    KH = KCH // 2
    W = NPAD // 16
    mesh = plsc.VectorSubcoreMesh(core_axis_name="c", subcore_axis_name="s")

    @functools.partial(
        pl.kernel,
        out_type=jax.ShapeDtypeStruct((2, 2, NPAD), jnp.float32),
        mesh=mesh,
        compiler_params=pltpu.CompilerParams(needs_layout_passes=False),
        scratch_types=[
            pltpu.VMEM((KH, 128), jnp.int32),
            pltpu.VMEM((KH, 128), jnp.int32),
            pltpu.VMEM((NPAD,), jnp.float32),
            pltpu.VMEM((NPAD,), jnp.float32),
            pltpu.VMEM((16 * W,), jnp.float32),
            pltpu.VMEM((W,), jnp.float32),
            pltpu.VMEM_SHARED((2 * 16 * NPAD,), jnp.float32),
        ],
    )
    def k1(src_hbm, dst_hbm, out_hbm, srcb, dstb, cntf, cntu, red, acc, shared):
        c = lax.axis_index("c")
        s = lax.axis_index("s")
        pltpu.sync_copy(src_hbm.at[s, pl.ds(c * KH, KH)], srcb)
        pltpu.sync_copy(dst_hbm.at[s, pl.ds(c * KH, KH)], dstb)

        zeros = jnp.zeros((L,), jnp.float32)

        def zero_body(i, _):
            cntf[pl.ds(i * L, L)] = zeros
            cntu[pl.ds(i * L, L)] = zeros
            return _

        lax.fori_loop(0, NPAD // L, zero_body, None)

        ones = jnp.ones((L,), jnp.float32)

        def cnt_body(k, _):
            for j in range(128 // L):
                sv = srcb[k, pl.ds(j * L, L)]
                dv = dstb[k, pl.ds(j * L, L)]
                plsc.addupdate_scatter(cntf, [dv], ones)
                plsc.addupdate_scatter(cntu, [sv], ones)
            return _

        lax.fori_loop(0, KH, cnt_body, None)

        pltpu.sync_copy(cntf, shared.at[pl.ds((2 * s) * NPAD, NPAD)])
        pltpu.sync_copy(cntu, shared.at[pl.ds((2 * s + 1) * NPAD, NPAD)])
        plsc.subcore_barrier()

        off = s * W
        for which in range(2):
            for t in range(16):
                pltpu.sync_copy(
                    shared.at[pl.ds((2 * t + which) * NPAD + off, W)],
                    red.at[pl.ds(t * W, W)])

            def red_body(j, _):
                v = red[pl.ds(j * L, L)]
                for t in range(1, 16):
                    v = v + red[pl.ds(t * W + j * L, L)]
                acc[pl.ds(j * L, L)] = v
                return _

            lax.fori_loop(0, W // L, red_body, None)
            pltpu.sync_copy(acc, out_hbm.at[c, which, pl.ds(off, W)])

    return k1


def _make_k3(NPAD, KCH, CW=128, NBUF=2):
    ZR = NPAD // 16
    KCH2 = KCH * 128 // CW
    NSPLIT = 2
    KH = KCH2 // NSPLIT
    mesh = plsc.VectorSubcoreMesh(core_axis_name="c", subcore_axis_name="s")

    @functools.partial(
        pl.kernel,
        out_type=jax.ShapeDtypeStruct((4, NPAD, 128), jnp.float32),
        mesh=mesh,
        compiler_params=pltpu.CompilerParams(needs_layout_passes=False),
        scratch_types=[
            pltpu.VMEM((KH, CW), jnp.int32),
            pltpu.VMEM((KH, CW), jnp.int32),
            [pltpu.VMEM((CW, 128), jnp.float32) for _ in range(NBUF)],
            [pltpu.SemaphoreType.DMA for _ in range(NBUF)],
            pltpu.VMEM_SHARED((NPAD, 128), jnp.float32),
        ],
    )
    def k3(table_hbm, gidx_hbm, sidx_hbm, out_hbm, gbuf, sbuf, rows, sems,
           acc):
        c = lax.axis_index("c")
        s = lax.axis_index("s")

        zeros = jnp.zeros((L,), jnp.float32)

        def zrow_body(i, _):
            for j in range(128 // L):
                rows[0][i, pl.ds(j * L, L)] = zeros
            return _

        for conv in range(2):
            lax.fori_loop(0, CW, zrow_body, None)
            for r in range(ZR // CW):
                pltpu.sync_copy(rows[0], acc.at[pl.ds(s * ZR + r * CW, CW)])
            plsc.subcore_barrier()

            for half in range(NSPLIT):
                pltpu.sync_copy(gidx_hbm.at[conv, c, s, pl.ds(half * KH, KH)],
                                gbuf)
                pltpu.sync_copy(sidx_hbm.at[conv, s, pl.ds(half * KH, KH)],
                                sbuf)
                for b in range(NBUF):
                    pltpu.async_copy(table_hbm.at[gbuf.at[b]], rows[b],
                                     sems[b])

                def outer_body(g, _):
                    for b in range(NBUF):
                        k = g * NBUF + b
                        pltpu.make_async_copy(table_hbm.at[gbuf.at[k]],
                                              rows[b], sems[b]).wait()
                        pltpu.sync_copy(rows[b], acc.at[sbuf.at[k]], add=True)

                        @pl.when(k + NBUF < KH)
                        def _():
                            pltpu.async_copy(table_hbm.at[gbuf.at[k + NBUF]],
                                             rows[b], sems[b])
                    return _

                lax.fori_loop(0, KH // NBUF, outer_body, None)
            plsc.subcore_barrier()
            pltpu.sync_copy(acc.at[pl.ds(s * ZR, ZR)],
                            out_hbm.at[2 * conv + c, pl.ds(s * ZR, ZR)])

    return k3


def _make_k5(NPAD, KCH, N):
    W = NPAD // 16
    mesh = plsc.VectorSubcoreMesh(core_axis_name="c", subcore_axis_name="s")

    @functools.partial(
        pl.kernel,
        out_type=jax.ShapeDtypeStruct((NPAD,), jnp.float32),
        mesh=mesh,
        compiler_params=pltpu.CompilerParams(needs_layout_passes=False),
        scratch_types=[
            pltpu.VMEM((KCH, 128), jnp.int32),
            pltpu.VMEM((KCH, 128), jnp.int32),
            pltpu.VMEM((NPAD,), jnp.float32),
            pltpu.VMEM((NPAD,), jnp.float32),
            pltpu.VMEM((16 * W,), jnp.float32),
            pltpu.VMEM((W,), jnp.float32),
            pltpu.VMEM((W,), jnp.float32),
            pltpu.VMEM((L,), jnp.float32),
            pltpu.VMEM_SHARED((16 * NPAD,), jnp.float32),
        ],
    )
    def k5(sp_hbm, dinv_hbm, src_hbm, dst_hbm, bo_hbm, out_hbm,
           srcb, dstb, spbuf, tbuf, red, dvb, obuf, bob, shared):
        c = lax.axis_index("c")
        s = lax.axis_index("s")

        @pl.when(c == 0)
        def _sc0():
            pltpu.sync_copy(sp_hbm, spbuf)
            pltpu.sync_copy(src_hbm.at[s], srcb)
            pltpu.sync_copy(dst_hbm.at[s], dstb)
            pltpu.sync_copy(bo_hbm, bob)

            zeros = jnp.zeros((L,), jnp.float32)

            def zero_body(i, _):
                tbuf[pl.ds(i * L, L)] = zeros
                return _

            lax.fori_loop(0, NPAD // L, zero_body, None)

            def scat_body(k, _):
                for j in range(128 // L):
                    sv = srcb[k, pl.ds(j * L, L)]
                    dv = dstb[k, pl.ds(j * L, L)]
                    g = plsc.load_gather(spbuf, [sv])
                    plsc.addupdate_scatter(tbuf, [dv], g)
                return _

            lax.fori_loop(0, KCH, scat_body, None)

            pltpu.sync_copy(tbuf, shared.at[pl.ds(s * NPAD, NPAD)])
            plsc.subcore_barrier()

            off = s * W
            for t in range(16):
                pltpu.sync_copy(shared.at[pl.ds(t * NPAD + off, W)],
                                red.at[pl.ds(t * W, W)])
            pltpu.sync_copy(dinv_hbm.at[pl.ds(off, W)], dvb)

            bo = bob[pl.ds(0, L)]
            one = jnp.ones((L,), jnp.float32)

            def fin_body(j, _):
                v = red[pl.ds(j * L, L)]
                for t in range(1, 16):
                    v = v + red[pl.ds(t * W + j * L, L)]
                z = dvb[pl.ds(j * L, L)] * (v + spbuf[pl.ds(off + j * L, L)]) + bo
                obuf[pl.ds(j * L, L)] = one / (one + jnp.exp(-z))
                return _

            lax.fori_loop(0, W // L, fin_body, None)
            pltpu.sync_copy(obuf, out_hbm.at[pl.ds(off, W)])

    return k5


def _k2_body(HID, x_ref, wt_ref, cf0_ref, cf1_ref, cu0_ref, cu1_ref,
             tab_ref, dvf_ref, dvu_ref):
    HC = HID // 2
    y = jnp.dot(x_ref[...], wt_ref[...], preferred_element_type=jnp.float32)
    dinvf = lax.rsqrt(cf0_ref[...] + cf1_ref[...] + 1.0)
    dinvu = lax.rsqrt(cu0_ref[...] + cu1_ref[...] + 1.0)
    hf = y[:, :HID] * dinvf
    hu = y[:, HID:] * dinvu
    tab_ref[0] = hf[:, :HC]
    tab_ref[1] = hf[:, HC:]
    tab_ref[2] = hu[:, :HC]
    tab_ref[3] = hu[:, HC:]
    dvf_ref[...] = dinvf
    dvu_ref[...] = dinvu


def _k4_body(N, B, seg_ref, tab_ref, dvf_ref, dvu_ref,
             bf_ref, bu_ref, wfc_ref, bfc_ref, wo_ref, sp_ref):
    dvf = dvf_ref[...]
    hf = jnp.concatenate([seg_ref[0] + tab_ref[0], seg_ref[1] + tab_ref[1]], axis=1)
    hf = jnp.maximum(dvf * hf + bf_ref[...], 0.0)
    hu = jnp.concatenate([seg_ref[2] + tab_ref[2], seg_ref[3] + tab_ref[3]], axis=1)
    hu = jnp.maximum(dvu_ref[...] * hu + bu_ref[...], 0.0)
    hcat = jnp.concatenate([hf, hu], axis=1)
    h2 = jnp.dot(hcat, wfc_ref[...], preferred_element_type=jnp.float32)
    h2 = jnp.maximum(h2 + bfc_ref[...], 0.0)
    sv = jnp.dot(h2, wo_ref[...], preferred_element_type=jnp.float32)
    row = pl.program_id(0) * B + lax.broadcasted_iota(jnp.int32, (B, 1), 0)
    sp_ref[...] = jnp.where(row < N, dvf * sv, 0.0)


def kernel(x, edge_index, Wf, bf, Wu, bu, Wfc, bfc, Wo, bo):
    N, IN = x.shape
    HID = Wf.shape[0]
    HC = HID // 2
    E = edge_index.shape[1]
    NPAD = _round_up(N + 8, 2048)
    EPAD = _round_up(E, 32 * 128)
    EPT = EPAD // 16
    KCH = EPT // 128
    B = 2048
    GRID = NPAD // B

    src = edge_index[0].astype(jnp.int32)
    dst = edge_index[1].astype(jnp.int32)
    pad = jnp.full((EPAD - E,), N, jnp.int32)
    srcp = jnp.concatenate([src, pad])
    dstp = jnp.concatenate([dst, pad])
    sr3 = srcp.reshape(16, KCH, 128)
    dr3 = dstp.reshape(16, KCH, 128)
    CW = 128
    gidx = jnp.stack([srcp, srcp + NPAD, dstp + 2 * NPAD, dstp + 3 * NPAD]
                     ).reshape(2, 2, 16, EPT // CW, CW)
    sidx = jnp.stack([dstp, srcp]).reshape(2, 16, EPT // CW, CW)

    xp = jnp.pad(x, ((0, NPAD - N), (0, 0)))
    WcatT = jnp.concatenate([Wf, Wu], axis=0).T
    WfcT = Wfc.T
    WoT = Wo.T

    counts = _make_k1(NPAD, KCH)(sr3, dr3)
    cf0 = counts[0, 0].reshape(NPAD, 1)
    cf1 = counts[1, 0].reshape(NPAD, 1)
    cu0 = counts[0, 1].reshape(NPAD, 1)
    cu1 = counts[1, 1].reshape(NPAD, 1)

    col = pl.BlockSpec((B, 1), lambda i: (i, 0))
    tab = pl.BlockSpec((4, B, HC), lambda i: (0, i, 0))
    yfu, dvf, dvu = pl.pallas_call(
        functools.partial(_k2_body, HID),
        grid=(GRID,),
        in_specs=[
            pl.BlockSpec((B, IN), lambda i: (i, 0)),
            pl.BlockSpec((IN, 2 * HID), lambda i: (0, 0)),
            col, col, col, col,
        ],
        out_specs=[tab, col, col],
        out_shape=[
            jax.ShapeDtypeStruct((4, NPAD, HC), jnp.float32),
            jax.ShapeDtypeStruct((NPAD, 1), jnp.float32),
            jax.ShapeDtypeStruct((NPAD, 1), jnp.float32),
        ],
    )(xp, WcatT, cf0, cf1, cu0, cu1)

    seg = _make_k3(NPAD, KCH, CW=CW, NBUF=2)(yfu.reshape(4 * NPAD, HC), gidx, sidx)

    sp = pl.pallas_call(
        functools.partial(_k4_body, N, B),
        grid=(GRID,),
        in_specs=[
            tab, tab, col, col,
            pl.BlockSpec((1, HID), lambda i: (0, 0)),
            pl.BlockSpec((1, HID), lambda i: (0, 0)),
            pl.BlockSpec((2 * HID, HID), lambda i: (0, 0)),
            pl.BlockSpec((1, HID), lambda i: (0, 0)),
            pl.BlockSpec((HID, 1), lambda i: (0, 0)),
        ],
        out_specs=col,
        out_shape=jax.ShapeDtypeStruct((NPAD, 1), jnp.float32),
    )(seg, yfu, dvf, dvu,
      bf.reshape(1, HID), bu.reshape(1, HID), WfcT, bfc.reshape(1, HID), WoT)

    bo16 = jnp.broadcast_to(bo, (L,)).astype(jnp.float32)
    out = _make_k5(NPAD, KCH, N)(
        sp.reshape(NPAD), dvf.reshape(NPAD), sr3, dr3, bo16)
    return out[:N].reshape(N, 1)
